# baseline (device time: 21779 ns/iter reference)
import functools

import jax
import jax.numpy as jnp
from jax import lax
from jax.experimental import pallas as pl
from jax.experimental.pallas import tpu as pltpu

N_DEV = 8
GRID = 8


def kernel(x):
    m, n = x.shape
    nb = n // GRID

    def body(x_hbm, x_ref, out_ref, halo_ref, send_sems, recv_sems):
        step = pl.program_id(0)
        my = lax.axis_index("i")
        has_left = my > 0
        has_right = my < N_DEV - 1

        send_left = pltpu.make_async_remote_copy(
            src_ref=x_hbm.at[pl.ds(0, 1)],
            dst_ref=halo_ref.at[1],
            send_sem=send_sems.at[0],
            recv_sem=recv_sems.at[1],
            device_id=(my - 1,),
            device_id_type=pl.DeviceIdType.MESH,
        )
        send_right = pltpu.make_async_remote_copy(
            src_ref=x_hbm.at[pl.ds(m - 1, 1)],
            dst_ref=halo_ref.at[0],
            send_sem=send_sems.at[1],
            recv_sem=recv_sems.at[0],
            device_id=(my + 1,),
            device_id_type=pl.DeviceIdType.MESH,
        )
        recv_from_left = pltpu.make_async_remote_copy(
            src_ref=x_hbm.at[pl.ds(0, 1)],
            dst_ref=halo_ref.at[0],
            send_sem=send_sems.at[0],
            recv_sem=recv_sems.at[0],
            device_id=(my,),
            device_id_type=pl.DeviceIdType.MESH,
        )
        recv_from_right = pltpu.make_async_remote_copy(
            src_ref=x_hbm.at[pl.ds(0, 1)],
            dst_ref=halo_ref.at[1],
            send_sem=send_sems.at[0],
            recv_sem=recv_sems.at[1],
            device_id=(my,),
            device_id_type=pl.DeviceIdType.MESH,
        )

        @pl.when(step == 0)
        def _():
            barrier = pltpu.get_barrier_semaphore()

            @pl.when(has_left)
            def _():
                pl.semaphore_signal(
                    barrier, inc=1, device_id=(my - 1,),
                    device_id_type=pl.DeviceIdType.MESH,
                )

            @pl.when(jnp.logical_not(has_left))
            def _():
                pl.semaphore_signal(barrier, inc=1)

            @pl.when(has_right)
            def _():
                pl.semaphore_signal(
                    barrier, inc=1, device_id=(my + 1,),
                    device_id_type=pl.DeviceIdType.MESH,
                )

            @pl.when(jnp.logical_not(has_right))
            def _():
                pl.semaphore_signal(barrier, inc=1)

            pl.semaphore_wait(barrier, 2)

            @pl.when(has_left)
            def _():
                send_left.start()

            @pl.when(has_right)
            def _():
                send_right.start()

            @pl.when(has_left)
            def _():
                recv_from_left.wait_recv()

            @pl.when(has_right)
            def _():
                recv_from_right.wait_recv()

        out_ref[pl.ds(1, m - 2), :] = (
            0.25 * x_ref[pl.ds(0, m - 2), :]
            + 0.5 * x_ref[pl.ds(1, m - 2), :]
            + 0.25 * x_ref[pl.ds(2, m - 2), :]
        ).astype(out_ref.dtype)

        col = pl.ds(step * nb, nb)

        @pl.when(has_left)
        def _():
            out_ref[pl.ds(0, 1), :] = (
                0.25 * halo_ref[0, :, col]
                + 0.5 * x_ref[pl.ds(0, 1), :]
                + 0.25 * x_ref[pl.ds(1, 1), :]
            ).astype(out_ref.dtype)

        @pl.when(jnp.logical_not(has_left))
        def _():
            out_ref[pl.ds(0, 1), :] = x_ref[pl.ds(0, 1), :].astype(out_ref.dtype)

        @pl.when(has_right)
        def _():
            out_ref[pl.ds(m - 1, 1), :] = (
                0.25 * x_ref[pl.ds(m - 2, 1), :]
                + 0.5 * x_ref[pl.ds(m - 1, 1), :]
                + 0.25 * halo_ref[1, :, col]
            ).astype(out_ref.dtype)

        @pl.when(jnp.logical_not(has_right))
        def _():
            out_ref[pl.ds(m - 1, 1), :] = x_ref[pl.ds(m - 1, 1), :].astype(
                out_ref.dtype
            )

        @pl.when(step == GRID - 1)
        def _():
            @pl.when(has_left)
            def _():
                send_left.wait_send()

            @pl.when(has_right)
            def _():
                send_right.wait_send()

            @functools.partial(
                pl.run_scoped, second_barrier=pltpu.SemaphoreType.REGULAR
            )
            def _(second_barrier):
                @pl.when(has_left)
                def _():
                    pl.semaphore_signal(
                        second_barrier, inc=1, device_id=(my - 1,),
                        device_id_type=pl.DeviceIdType.MESH,
                    )

                @pl.when(jnp.logical_not(has_left))
                def _():
                    pl.semaphore_signal(second_barrier, inc=1)

                @pl.when(has_right)
                def _():
                    pl.semaphore_signal(
                        second_barrier, inc=1, device_id=(my + 1,),
                        device_id_type=pl.DeviceIdType.MESH,
                    )

                @pl.when(jnp.logical_not(has_right))
                def _():
                    pl.semaphore_signal(second_barrier, inc=1)

                pl.semaphore_wait(second_barrier, 2)

    return pl.pallas_call(
        body,
        grid=(GRID,),
        out_shape=jax.ShapeDtypeStruct((m, n), jnp.bfloat16),
        in_specs=[
            pl.BlockSpec(memory_space=pl.ANY),
            pl.BlockSpec((m, nb), lambda j: (0, j)),
        ],
        out_specs=pl.BlockSpec((m, nb), lambda j: (0, j)),
        scratch_shapes=[
            pltpu.VMEM((2, 1, n), x.dtype),
            pltpu.SemaphoreType.DMA((2,)),
            pltpu.SemaphoreType.DMA((2,)),
        ],
        compiler_params=pltpu.CompilerParams(
            collective_id=0,
            dimension_semantics=("arbitrary",),
        ),
    )(x, x)


# device time: 17932 ns/iter; 1.2145x vs baseline; 1.2145x over previous
import jax
import jax.numpy as jnp
from jax import lax
from jax.experimental import pallas as pl
from jax.experimental.pallas import tpu as pltpu

N_DEV = 8
C = 8


def kernel(x):
    m, n = x.shape
    mc = m // C

    def body(x_ref, o_hbm, halo_ref, obuf, s_ref,
             out_sems, send_sems, recv_sems):
        my = lax.axis_index("i")
        has_left = my > 0
        has_right = my < N_DEV - 1

        barrier = pltpu.get_barrier_semaphore()

        @pl.when(has_left)
        def _():
            pl.semaphore_signal(
                barrier, inc=1, device_id=(my - 1,),
                device_id_type=pl.DeviceIdType.MESH,
            )

        @pl.when(jnp.logical_not(has_left))
        def _():
            pl.semaphore_signal(barrier, inc=1)

        @pl.when(has_right)
        def _():
            pl.semaphore_signal(
                barrier, inc=1, device_id=(my + 1,),
                device_id_type=pl.DeviceIdType.MESH,
            )

        @pl.when(jnp.logical_not(has_right))
        def _():
            pl.semaphore_signal(barrier, inc=1)

        pl.semaphore_wait(barrier, 2)

        send_left = pltpu.make_async_remote_copy(
            src_ref=x_ref.at[pl.ds(0, 1)],
            dst_ref=halo_ref.at[1],
            send_sem=send_sems.at[0],
            recv_sem=recv_sems.at[1],
            device_id=(my - 1,),
            device_id_type=pl.DeviceIdType.MESH,
        )
        send_right = pltpu.make_async_remote_copy(
            src_ref=x_ref.at[pl.ds(m - 1, 1)],
            dst_ref=halo_ref.at[0],
            send_sem=send_sems.at[1],
            recv_sem=recv_sems.at[0],
            device_id=(my + 1,),
            device_id_type=pl.DeviceIdType.MESH,
        )
        recv_from_left = pltpu.make_async_remote_copy(
            src_ref=x_ref.at[pl.ds(0, 1)],
            dst_ref=halo_ref.at[0],
            send_sem=send_sems.at[0],
            recv_sem=recv_sems.at[0],
            device_id=(my,),
            device_id_type=pl.DeviceIdType.MESH,
        )
        recv_from_right = pltpu.make_async_remote_copy(
            src_ref=x_ref.at[pl.ds(0, 1)],
            dst_ref=halo_ref.at[1],
            send_sem=send_sems.at[0],
            recv_sem=recv_sems.at[1],
            device_id=(my,),
            device_id_type=pl.DeviceIdType.MESH,
        )

        @pl.when(has_left)
        def _():
            send_left.start()

        @pl.when(has_right)
        def _():
            send_right.start()

        pending = [None, None]

        def stencil_rows(a, j0, j1, slot):
            cnt = j1 + 1 - j0
            s_ref[pl.ds(j0, cnt), :] = (
                x_ref[pl.ds(a - 1 + j0, cnt), :]
                + x_ref[pl.ds(a + j0, cnt), :]
            )
            obuf[slot, pl.ds(j0, j1 - j0), :] = (
                0.25 * (s_ref[pl.ds(j0, j1 - j0), :]
                        + s_ref[pl.ds(j0 + 1, j1 - j0), :])
            ).astype(obuf.dtype)

        def flush(c, slot):
            cp = pltpu.make_async_copy(
                obuf.at[slot],
                o_hbm.at[pl.ds(c * mc, mc)],
                out_sems.at[slot],
            )
            cp.start()
            pending[slot] = cp

        order = list(range(1, C - 1)) + [0, C - 1]
        for k, c in enumerate(order):
            slot = k % 2
            if pending[slot] is not None:
                pending[slot].wait()
            if c == 0:
                stencil_rows(0, 1, mc, slot)

                @pl.when(has_left)
                def _():
                    recv_from_left.wait_recv()
                    obuf[slot, pl.ds(0, 1), :] = (
                        0.25 * halo_ref[0]
                        + 0.5 * x_ref[pl.ds(0, 1), :]
                        + 0.25 * x_ref[pl.ds(1, 1), :]
                    ).astype(obuf.dtype)

                @pl.when(jnp.logical_not(has_left))
                def _():
                    obuf[slot, pl.ds(0, 1), :] = x_ref[
                        pl.ds(0, 1), :
                    ].astype(obuf.dtype)

            elif c == C - 1:
                a = c * mc
                stencil_rows(a, 0, mc - 1, slot)

                @pl.when(has_right)
                def _():
                    recv_from_right.wait_recv()
                    obuf[slot, pl.ds(mc - 1, 1), :] = (
                        0.25 * x_ref[pl.ds(m - 2, 1), :]
                        + 0.5 * x_ref[pl.ds(m - 1, 1), :]
                        + 0.25 * halo_ref[1]
                    ).astype(obuf.dtype)

                @pl.when(jnp.logical_not(has_right))
                def _():
                    obuf[slot, pl.ds(mc - 1, 1), :] = x_ref[
                        pl.ds(m - 1, 1), :
                    ].astype(obuf.dtype)

            else:
                stencil_rows(c * mc, 0, mc, slot)
            flush(c, slot)

        pending[0].wait()
        pending[1].wait()

        @pl.when(has_left)
        def _():
            send_left.wait_send()

        @pl.when(has_right)
        def _():
            send_right.wait_send()

    return pl.pallas_call(
        body,
        out_shape=jax.ShapeDtypeStruct((m, n), jnp.bfloat16),
        in_specs=[pl.BlockSpec(memory_space=pltpu.MemorySpace.VMEM)],
        out_specs=pl.BlockSpec(memory_space=pl.ANY),
        scratch_shapes=[
            pltpu.VMEM((2, 1, n), x.dtype),
            pltpu.VMEM((2, mc, n), jnp.bfloat16),
            pltpu.VMEM((mc + 1, n), x.dtype),
            pltpu.SemaphoreType.DMA((2,)),
            pltpu.SemaphoreType.DMA((2,)),
            pltpu.SemaphoreType.DMA((2,)),
        ],
        compiler_params=pltpu.CompilerParams(collective_id=0),
    )(x)


# device time: 14365 ns/iter; 1.5161x vs baseline; 1.2483x over previous
import jax
import jax.numpy as jnp
from jax import lax
from jax.experimental import pallas as pl
from jax.experimental.pallas import tpu as pltpu

N_DEV = 8
C = 8
OV = 8


def kernel(x):
    m, n = x.shape
    mc = m // C

    def in_rows(c):
        lo = 0 if c == 0 else c * mc - OV
        hi = m if c == C - 1 else c * mc + mc + OV
        return lo, hi - lo

    def body(x_hbm, o_hbm, halo_ref, xbuf, obuf, s_ref,
             in_sems, out_sems, send_sems, recv_sems):
        my = lax.axis_index("i")
        has_left = my > 0
        has_right = my < N_DEV - 1

        barrier = pltpu.get_barrier_semaphore()

        @pl.when(has_left)
        def _():
            pl.semaphore_signal(
                barrier, inc=1, device_id=(my - 1,),
                device_id_type=pl.DeviceIdType.MESH,
            )

        @pl.when(jnp.logical_not(has_left))
        def _():
            pl.semaphore_signal(barrier, inc=1)

        @pl.when(has_right)
        def _():
            pl.semaphore_signal(
                barrier, inc=1, device_id=(my + 1,),
                device_id_type=pl.DeviceIdType.MESH,
            )

        @pl.when(jnp.logical_not(has_right))
        def _():
            pl.semaphore_signal(barrier, inc=1)

        pl.semaphore_wait(barrier, 2)

        send_left = pltpu.make_async_remote_copy(
            src_ref=x_hbm.at[pl.ds(0, 1)],
            dst_ref=halo_ref.at[1],
            send_sem=send_sems.at[0],
            recv_sem=recv_sems.at[1],
            device_id=(my - 1,),
            device_id_type=pl.DeviceIdType.MESH,
        )
        send_right = pltpu.make_async_remote_copy(
            src_ref=x_hbm.at[pl.ds(m - 1, 1)],
            dst_ref=halo_ref.at[0],
            send_sem=send_sems.at[1],
            recv_sem=recv_sems.at[0],
            device_id=(my + 1,),
            device_id_type=pl.DeviceIdType.MESH,
        )
        recv_from_left = pltpu.make_async_remote_copy(
            src_ref=x_hbm.at[pl.ds(0, 1)],
            dst_ref=halo_ref.at[0],
            send_sem=send_sems.at[0],
            recv_sem=recv_sems.at[0],
            device_id=(my,),
            device_id_type=pl.DeviceIdType.MESH,
        )
        recv_from_right = pltpu.make_async_remote_copy(
            src_ref=x_hbm.at[pl.ds(0, 1)],
            dst_ref=halo_ref.at[1],
            send_sem=send_sems.at[0],
            recv_sem=recv_sems.at[1],
            device_id=(my,),
            device_id_type=pl.DeviceIdType.MESH,
        )

        @pl.when(has_left)
        def _():
            send_left.start()

        @pl.when(has_right)
        def _():
            send_right.start()

        order = list(range(1, C - 1)) + [0, C - 1]

        in_pending = {}

        def start_in(c, slot):
            lo, cnt = in_rows(c)
            cp = pltpu.make_async_copy(
                x_hbm.at[pl.ds(lo, cnt)],
                xbuf.at[slot, pl.ds(0, cnt)],
                in_sems.at[slot],
            )
            cp.start()
            in_pending[c] = cp

        start_in(order[0], 0)
        start_in(order[1], 1)

        out_pending = [None, None]
        for k, c in enumerate(order):
            islot = k % 3
            oslot = k % 2
            if k + 2 < C:
                start_in(order[k + 2], (k + 2) % 3)
            in_pending[c].wait()
            a = c * mc
            off = 0 if c == 0 else OV
            j0 = 1 if c == 0 else 0
            j1 = mc - 1 if c == C - 1 else mc
            cnt = j1 + 1 - j0
            s_ref[pl.ds(j0, cnt), :] = (
                xbuf[islot, pl.ds(off - 1 + j0, cnt), :]
                + xbuf[islot, pl.ds(off + j0, cnt), :]
            )
            if out_pending[oslot] is not None:
                out_pending[oslot].wait()
            obuf[oslot, pl.ds(j0, j1 - j0), :] = (
                0.25 * (s_ref[pl.ds(j0, j1 - j0), :]
                        + s_ref[pl.ds(j0 + 1, j1 - j0), :])
            ).astype(obuf.dtype)

            if c == 0:
                @pl.when(has_left)
                def _():
                    recv_from_left.wait_recv()
                    obuf[oslot, pl.ds(0, 1), :] = (
                        0.25 * halo_ref[0]
                        + 0.5 * xbuf[islot, pl.ds(0, 1), :]
                        + 0.25 * xbuf[islot, pl.ds(1, 1), :]
                    ).astype(obuf.dtype)

                @pl.when(jnp.logical_not(has_left))
                def _():
                    obuf[oslot, pl.ds(0, 1), :] = xbuf[
                        islot, pl.ds(0, 1), :
                    ].astype(obuf.dtype)

            if c == C - 1:
                @pl.when(has_right)
                def _():
                    recv_from_right.wait_recv()
                    obuf[oslot, pl.ds(mc - 1, 1), :] = (
                        0.25 * xbuf[islot, pl.ds(off + mc - 2, 1), :]
                        + 0.5 * xbuf[islot, pl.ds(off + mc - 1, 1), :]
                        + 0.25 * halo_ref[1]
                    ).astype(obuf.dtype)

                @pl.when(jnp.logical_not(has_right))
                def _():
                    obuf[oslot, pl.ds(mc - 1, 1), :] = xbuf[
                        islot, pl.ds(off + mc - 1, 1), :
                    ].astype(obuf.dtype)

            cp = pltpu.make_async_copy(
                obuf.at[oslot],
                o_hbm.at[pl.ds(a, mc)],
                out_sems.at[oslot],
            )
            cp.start()
            out_pending[oslot] = cp

        out_pending[0].wait()
        out_pending[1].wait()

        @pl.when(has_left)
        def _():
            send_left.wait_send()

        @pl.when(has_right)
        def _():
            send_right.wait_send()

    return pl.pallas_call(
        body,
        out_shape=jax.ShapeDtypeStruct((m, n), jnp.bfloat16),
        in_specs=[pl.BlockSpec(memory_space=pltpu.MemorySpace.HBM)],
        out_specs=pl.BlockSpec(memory_space=pltpu.MemorySpace.HBM),
        scratch_shapes=[
            pltpu.VMEM((2, 1, n), x.dtype),
            pltpu.VMEM((3, mc + 2 * OV, n), x.dtype),
            pltpu.VMEM((2, mc, n), jnp.bfloat16),
            pltpu.VMEM((mc + 1, n), x.dtype),
            pltpu.SemaphoreType.DMA((3,)),
            pltpu.SemaphoreType.DMA((2,)),
            pltpu.SemaphoreType.DMA((2,)),
            pltpu.SemaphoreType.DMA((2,)),
        ],
        compiler_params=pltpu.CompilerParams(collective_id=0),
    )(x)
